# baseline (speedup 1.0000x reference)
import functools

import jax
import jax.numpy as jnp
from jax import lax
from jax.experimental import pallas as pl
from jax.experimental.pallas import tpu as pltpu
from jax.experimental.pallas import tpu_sc as plsc

D = 64
B = 1024
S = 200
NC, NS = 2, 16
NW = NC * NS
NBB = B // 128
NCB = S * NBB // NW
LANES = 16
KD = D // LANES

_mesh = plsc.VectorSubcoreMesh(core_axis_name="c", subcore_axis_name="s")


@functools.partial(
    pl.kernel,
    out_type=jax.ShapeDtypeStruct((S, D // 8, NBB, 8, 128), jnp.float32),
    mesh=_mesh,
    scratch_types=[
        pltpu.VMEM((S, 128), jnp.int32),
        pltpu.VMEM((3, 128, 128), jnp.float32),
        pltpu.VMEM((2, D, 128), jnp.float32),
        pltpu.VMEM((D, 256), jnp.float32),
        pltpu.SemaphoreType.DMA((3,)),
        pltpu.SemaphoreType.DMA((2,)),
    ],
    compiler_params=pltpu.CompilerParams(
        use_tc_tiling_on_sc=True, needs_layout_passes=False),
)
def _emb_lookup(ids_hbm, table_hbm, post_hbm, out_hbm, ids_v, gbuf,
                tstage, posT_v, gsem, ssem):
    wid = lax.axis_index("s") * NC + lax.axis_index("c")
    bb = lax.rem(wid, NBB)
    sq = wid // NBB
    pltpu.sync_copy(post_hbm.at[:, pl.ds(0, 256)], posT_v)
    pltpu.sync_copy(ids_hbm.at[:, pl.ds(bb * 128, 128)], ids_v)

    rows_c = [lax.iota(jnp.int32, 16) + 16 * bq for bq in range(8)]

    def fire_gather(k):
        b3 = lax.rem(k, 3)
        pltpu.async_copy(table_hbm.at[ids_v.at[sq + 4 * k]],
                         gbuf.at[b3], gsem.at[b3])

    def wait_gather(k):
        b3 = lax.rem(k, 3)
        pltpu.make_async_copy(table_hbm.at[ids_v.at[sq + 4 * k]],
                              gbuf.at[b3], gsem.at[b3]).wait()

    def fire_out(k):
        tb = lax.rem(k, 2)
        s = sq + 4 * k
        for r in range(8):
            pltpu.async_copy(tstage.at[tb, pl.ds(8 * r, 8)],
                             out_hbm.at[s, r, bb], ssem.at[tb])

    def wait_out(k):
        tb = lax.rem(k, 2)
        s = sq + 4 * k
        for r in range(8):
            pltpu.make_async_copy(tstage.at[tb, pl.ds(8 * r, 8)],
                                  out_hbm.at[s, r, bb], ssem.at[tb]).wait()

    fire_gather(0)
    fire_gather(1)

    def body(k, carry):
        b3 = lax.rem(k, 3)
        tb = lax.rem(k, 2)
        s = sq + 4 * k

        @pl.when(k < NCB - 2)
        def _():
            fire_gather(k + 2)

        wait_gather(k)

        @pl.when(k >= 2)
        def _():
            wait_out(k - 2)

        scol = jnp.full((16,), s, jnp.int32)
        prow = [plsc.load_gather(posT_v, [rows_c[j], scol]) for j in range(KD)]
        for d in range(D):
            p = jnp.full((16,), prow[d // 16][d % 16], jnp.float32)
            col = jnp.full((16,), d, jnp.int32)
            for bq in range(8):
                v = plsc.load_gather(gbuf.at[b3], [rows_c[bq], col])
                tstage[tb, d, pl.ds(16 * bq, 16)] = v + p

        fire_out(k)
        return carry

    lax.fori_loop(0, NCB, body, 0)
    wait_out(NCB - 2)
    wait_out(NCB - 1)


def kernel(token_ids, token_table, pos_table):
    ids_t = token_ids.T.astype(jnp.int32)
    table2 = jnp.pad(token_table, ((0, 0), (0, 64)))
    pos_t = pos_table.T
    out5 = _emb_lookup(ids_t, table2, pos_t)
    return out5.transpose(2, 4, 0, 1, 3).reshape(B, S, D)
